# baseline (speedup 1.0000x reference)
import jax
import jax.numpy as jnp
from jax.experimental import pallas as pl

MIN_SCORE = 0.1
MAX_OVERLAP = 0.45
CAND = 512
P_PAD = 3200


def _decode_body(locs_ref, priors_ref, out_ref):
    l0 = locs_ref[:, 0, :]
    l1 = locs_ref[:, 1, :]
    l2 = locs_ref[:, 2, :]
    l3 = locs_ref[:, 3, :]
    p0 = priors_ref[0:1, :]
    p1 = priors_ref[1:2, :]
    p2 = priors_ref[2:3, :]
    p3 = priors_ref[3:4, :]
    cx = l0 * p2 / 10.0 + p0
    cy = l1 * p3 / 10.0 + p1
    w = jnp.exp(l2 / 5.0) * p2
    h = jnp.exp(l3 / 5.0) * p3
    out_ref[:, 0, :] = cx - w / 2.0
    out_ref[:, 1, :] = cy - h / 2.0
    out_ref[:, 2, :] = cx + w / 2.0
    out_ref[:, 3, :] = cy + h / 2.0


def _decode_boxes(predicted_locs, priors):
    B, P, _ = predicted_locs.shape
    locs_t = jnp.transpose(predicted_locs, (0, 2, 1))
    locs_t = jnp.pad(locs_t, ((0, 0), (0, 4), (0, P_PAD - P)))
    priors_t = jnp.pad(jnp.transpose(priors, (1, 0)), ((0, 4), (0, P_PAD - P)))
    out = pl.pallas_call(
        _decode_body,
        out_shape=jax.ShapeDtypeStruct((B, 4, P_PAD), jnp.float32),
    )(locs_t, priors_t)
    return jnp.transpose(out[:, :, :P], (0, 2, 1))


def _pairwise_iou(a, b):
    lt = jnp.maximum(a[:, None, :2], b[None, :, :2])
    rb = jnp.minimum(a[:, None, 2:], b[None, :, 2:])
    wh = jnp.clip(rb - lt, 0.0)
    inter = wh[..., 0] * wh[..., 1]
    area_a = (a[:, 2] - a[:, 0]) * (a[:, 3] - a[:, 1])
    area_b = (b[:, 2] - b[:, 0]) * (b[:, 3] - b[:, 1])
    union = area_a[:, None] + area_b[None, :] - inter
    return inter / jnp.maximum(union, 1e-10)


def _nms_class(scores_c, decoded):
    masked = jnp.where(scores_c > MIN_SCORE, scores_c, -1.0)
    vals, idx = jax.lax.top_k(masked, CAND)
    boxes = decoded[idx]
    valid = vals > MIN_SCORE
    ov = _pairwise_iou(boxes, boxes)
    sup0 = jnp.logical_not(valid)

    def body(i, sup):
        cond = jnp.logical_and(jnp.logical_not(sup[i]), valid[i])
        merged = jnp.logical_or(sup, ov[i] > MAX_OVERLAP).at[i].set(False)
        return jnp.where(cond, merged, sup)

    sup = jax.lax.fori_loop(0, CAND, body, sup0)
    keep = jnp.logical_and(jnp.logical_not(sup), valid)
    return jnp.where(keep, vals, 0.0), boxes


def _detect_image(locs_i, probs_i, decoded, top_k):
    n_classes = probs_i.shape[1]
    kept_scores, cand_boxes = jax.vmap(lambda cs: _nms_class(cs, decoded))(probs_i[:, 1:].T)
    labels = jnp.broadcast_to(jnp.arange(1, n_classes)[:, None], kept_scores.shape)
    flat_s = kept_scores.reshape(-1)
    flat_b = cand_boxes.reshape(-1, 4)
    flat_l = labels.reshape(-1)
    flat_s = flat_s + (jnp.asarray(top_k) * 0).astype(flat_s.dtype)
    top_vals, top_idx = jax.lax.top_k(flat_s, 100)
    return flat_b[top_idx], flat_l[top_idx], top_vals


def kernel(predicted_locs, predicted_scores, priors_cxcy, top_k):
    probs = jax.nn.softmax(predicted_scores, axis=2)
    decoded = _decode_boxes(predicted_locs, priors_cxcy)
    return jax.vmap(
        lambda l, p, d: _detect_image(l, p, d, top_k)
    )(predicted_locs, probs, decoded)


# speedup vs baseline: 2.9344x; 2.9344x over previous
import jax
import jax.numpy as jnp
from jax.experimental import pallas as pl

MIN_SCORE = 0.1
MAX_OVERLAP = 0.45
CAND = 512
NCLS = 19
P = 3106
PP = 3328
OUTK = 128
K2 = 100
DN = (((1,), (1,)), ((), ()))


def _roll_cumsum(x, width, lane_iota):
    k = 1
    while k < width:
        x = x + jnp.where(lane_iota >= k, jnp.roll(x, k, axis=1), 0.0)
        k *= 2
    return x


def _detect_body(probs_ref, locs_ref, priors_ref, out_ref):
    f32 = jnp.float32
    probs = probs_ref[0]
    locs = locs_ref[0]
    pri = priors_ref[...]

    cx = locs[0:1] * pri[2:3] / 10.0 + pri[0:1]
    cy = locs[1:2] * pri[3:4] / 10.0 + pri[1:2]
    w = jnp.exp(locs[2:3] / 5.0) * pri[2:3]
    h = jnp.exp(locs[3:4] / 5.0) * pri[3:4]
    dx0 = cx - w / 2.0
    dy0 = cy - h / 2.0
    dx1 = cx + w / 2.0
    dy1 = cy + h / 2.0

    lane_p = jax.lax.broadcasted_iota(jnp.int32, (1, PP), 1)
    masked = jnp.where(lane_p < P,
                       jnp.where(probs > MIN_SCORE, probs, -1.0),
                       -2.0)

    def bisect1(_, lohi):
        lo, hi = lohi
        mid = (lo + hi) * 0.5
        cnt = jnp.sum(jnp.where(masked > mid, 1.0, 0.0), axis=1,
                      keepdims=True)
        take = cnt >= CAND
        return (jnp.where(take, mid, lo), jnp.where(take, hi, mid))

    _, t1 = jax.lax.fori_loop(0, 40, bisect1,
                              (jnp.full((NCLS, 1), -3.0, f32),
                               jnp.full((NCLS, 1), 2.0, f32)))

    gt1 = masked > t1
    gt1f = jnp.where(gt1, 1.0, 0.0)
    eq1f = jnp.where(masked == t1, 1.0, 0.0)
    n_gt1 = jnp.sum(gt1f, axis=1, keepdims=True)
    pre_gt1 = _roll_cumsum(gt1f, PP, lane_p) - gt1f
    pre_eq1 = _roll_cumsum(eq1f, PP, lane_p) - eq1f
    dest1 = jnp.where(gt1, pre_gt1,
                      jnp.where(masked == t1, n_gt1 + pre_eq1, 9999.0))

    q_iota = jax.lax.broadcasted_iota(jnp.int32, (CAND, PP), 0).astype(f32)
    idxrow = lane_p.astype(f32)
    comp = []
    for c in range(NCLS):
        oh = jnp.where(dest1[c:c + 1] == q_iota, 1.0, 0.0)
        dc = jnp.concatenate([masked[c:c + 1], dx0, dy0, dx1, dy1, idxrow],
                             axis=0)
        comp.append(jax.lax.dot_general(dc, oh, DN,
                                        preferred_element_type=f32,
                                        precision=jax.lax.Precision.HIGHEST))

    cval = jnp.concatenate([g[0:1] for g in comp], axis=0)
    cidx = jnp.concatenate([g[5:6] for g in comp], axis=0)

    v_p = cval[:, :, None]
    i_p = cidx[:, :, None]
    pq = jax.lax.broadcasted_iota(jnp.int32, (1, CAND, 1), 1)
    ngt3 = n_gt1[:, :, None]
    rank_blocks = []
    for o in range(0, CAND, OUTK):
        vq = cval[:, None, o:o + OUTK]
        iq = cidx[:, None, o:o + OUTK]
        beat = ((pq.astype(f32) < ngt3)
                & ((v_p > vq) | ((v_p == vq) & (i_p < iq))))
        rank_blocks.append(
            jnp.sum(jnp.where(beat, 1.0, 0.0), axis=1, keepdims=True))
    rankc3 = jnp.concatenate(rank_blocks, axis=2)

    s_lane = jax.lax.broadcasted_iota(jnp.int32, (1, CAND), 1)
    r_iota = jax.lax.broadcasted_iota(jnp.int32, (CAND, CAND), 0).astype(f32)
    svals_l, sx0_l, sy0_l, sx1_l, sy1_l = [], [], [], [], []
    for c in range(NCLS):
        tq = jnp.where(s_lane < n_gt1[c:c + 1], rankc3[c],
                       s_lane.astype(f32))
        oh = jnp.where(tq == r_iota, 1.0, 0.0)
        g = jax.lax.dot_general(comp[c][0:5], oh, DN,
                                preferred_element_type=f32,
                                        precision=jax.lax.Precision.HIGHEST)
        svals_l.append(g[0:1])
        sx0_l.append(g[1:2])
        sy0_l.append(g[2:3])
        sx1_l.append(g[3:4])
        sy1_l.append(g[4:5])
    svals = jnp.concatenate(svals_l, axis=0)
    sx0 = jnp.concatenate(sx0_l, axis=0)
    sy0 = jnp.concatenate(sy0_l, axis=0)
    sx1 = jnp.concatenate(sx1_l, axis=0)
    sy1 = jnp.concatenate(sy1_l, axis=0)

    valid = svals > MIN_SCORE
    validf = jnp.where(valid, 1.0, 0.0)
    area = (sx1 - sx0) * (sy1 - sy0)

    def body(i, supf):
        oh_i = jnp.where(s_lane == i, 1.0, 0.0)

        def ext(a):
            return jnp.sum(a * oh_i, axis=1, keepdims=True)

        bx0, by0, bx1, by1 = ext(sx0), ext(sy0), ext(sx1), ext(sy1)
        sup_i = ext(supf)
        val_i = ext(validf)
        cond = (sup_i == 0.0) & (val_i > 0.0)
        ltx = jnp.maximum(sx0, bx0)
        lty = jnp.maximum(sy0, by0)
        rbx = jnp.minimum(sx1, bx1)
        rby = jnp.minimum(sy1, by1)
        inter = jnp.clip(rbx - ltx, 0.0) * jnp.clip(rby - lty, 0.0)
        union = ext(area) + area - inter
        ov_row = inter / jnp.maximum(union, 1e-10)
        merged = jnp.maximum(supf, jnp.where(ov_row > MAX_OVERLAP, 1.0, 0.0))
        merged = merged * jnp.where(s_lane != i, 1.0, 0.0)
        return jnp.where(cond, merged, supf)

    supf = jax.lax.fori_loop(0, CAND, body, 1.0 - validf)
    kept = jnp.where((supf == 0.0) & valid, svals, 0.0)

    def bisect2(_, lohi):
        lo, hi = lohi
        mid = (lo + hi) * 0.5
        cnt = jnp.sum(jnp.where(kept > mid, 1.0, 0.0),
                      axis=(0, 1), keepdims=True)
        take = cnt >= K2
        return (jnp.where(take, mid, lo), jnp.where(take, hi, mid))

    _, t2 = jax.lax.fori_loop(0, 40, bisect2,
                              (jnp.full((1, 1), -1.0, f32),
                               jnp.full((1, 1), 2.0, f32)))
    t2 = jnp.where(t2 > MIN_SCORE, t2, 0.0)

    gt2 = kept > t2
    gt2f = jnp.where(gt2, 1.0, 0.0)
    eq2f = jnp.where(kept == t2, 1.0, 0.0)
    n_gt2 = jnp.sum(gt2f, axis=(0, 1), keepdims=True)

    tri = jnp.where(
        jax.lax.broadcasted_iota(jnp.int32, (NCLS, NCLS), 0)
        > jax.lax.broadcasted_iota(jnp.int32, (NCLS, NCLS), 1), 1.0, 0.0)

    def flat_prefix(mask):
        inc = _roll_cumsum(mask, CAND, s_lane)
        rowtot = inc[:, CAND - 1:CAND]
        rowpre = jax.lax.dot_general(tri, rowtot, (((1,), (0,)), ((), ())),
                                     preferred_element_type=f32,
                                        precision=jax.lax.Precision.HIGHEST)
        return inc - mask + rowpre

    dest2 = jnp.where(gt2, flat_prefix(gt2f),
                      jnp.where(kept == t2,
                                float(OUTK) + flat_prefix(eq2f), 9999.0))

    labrow = (jax.lax.broadcasted_iota(jnp.int32, (NCLS, CAND), 0)
              + 1).astype(f32)
    flatidx = (jax.lax.broadcasted_iota(jnp.int32, (NCLS, CAND), 0) * CAND
               + s_lane).astype(f32)
    q2_iota = jax.lax.broadcasted_iota(jnp.int32, (2 * OUTK, CAND), 0)
    q2_iota = q2_iota.astype(f32)
    acc = jnp.zeros((7, 2 * OUTK), f32)
    for c in range(NCLS):
        oh = jnp.where(dest2[c:c + 1] == q2_iota, 1.0, 0.0)
        dc = jnp.concatenate(
            [kept[c:c + 1], sx0[c:c + 1], sy0[c:c + 1], sx1[c:c + 1],
             sy1[c:c + 1], labrow[c:c + 1], flatidx[c:c + 1]], axis=0)
        acc = acc + jax.lax.dot_general(dc, oh, DN,
                                        preferred_element_type=f32,
                                        precision=jax.lax.Precision.HIGHEST)

    cgt = acc[:, :OUTK]
    cv = cgt[0:1]
    cf = cgt[6:7]
    cvt = jnp.transpose(cv)
    cft = jnp.transpose(cf)
    q128s = jax.lax.broadcasted_iota(jnp.int32, (OUTK, 1), 0)
    q128l = jax.lax.broadcasted_iota(jnp.int32, (1, OUTK), 1)
    beat2 = ((q128s.astype(f32) < n_gt2)
             & ((cvt > cv) | ((cvt == cv) & (cft < cf))))
    ranka = jnp.sum(jnp.where(beat2, 1.0, 0.0), axis=0, keepdims=True)
    rankc = jnp.where(q128l.astype(f32) < n_gt2, ranka, 9999.0)

    tq2 = jnp.concatenate([rankc, n_gt2 + q128l.astype(f32)], axis=1)
    oh3 = jnp.where(tq2 == jax.lax.broadcasted_iota(
        jnp.int32, (OUTK, 2 * OUTK), 0).astype(f32), 1.0, 0.0)
    outm = jax.lax.dot_general(acc, oh3, DN,
                               preferred_element_type=f32,
                                        precision=jax.lax.Precision.HIGHEST)
    out_ref[0] = jnp.concatenate([outm[0:6], jnp.zeros((1, OUTK), f32)],
                                 axis=0)


def kernel(predicted_locs, predicted_scores, priors_cxcy, top_k):
    B, p_cnt, _ = predicted_locs.shape
    probs = jax.nn.softmax(predicted_scores, axis=2)
    probs_t = jnp.transpose(probs, (0, 2, 1))[:, 1:, :]
    probs_t = jnp.pad(probs_t, ((0, 0), (0, 0), (0, PP - p_cnt)))
    locs_t = jnp.pad(jnp.transpose(predicted_locs, (0, 2, 1)),
                     ((0, 0), (0, 0), (0, PP - p_cnt)))
    priors_t = jnp.pad(jnp.transpose(priors_cxcy, (1, 0)),
                       ((0, 0), (0, PP - p_cnt)))

    res = pl.pallas_call(
        _detect_body,
        grid=(B,),
        in_specs=[
            pl.BlockSpec((1, NCLS, PP), lambda b: (b, 0, 0)),
            pl.BlockSpec((1, 4, PP), lambda b: (b, 0, 0)),
            pl.BlockSpec((4, PP), lambda b: (0, 0)),
        ],
        out_specs=pl.BlockSpec((1, 7, OUTK), lambda b: (b, 0, 0)),
        out_shape=jax.ShapeDtypeStruct((B, 7, OUTK), jnp.float32),
    )(probs_t, locs_t, priors_t)

    scores = res[:, 0, :K2] + (jnp.asarray(top_k) * 0).astype(jnp.float32)
    boxes = jnp.transpose(res[:, 1:5, :K2], (0, 2, 1))
    labels = res[:, 5, :K2].astype(jnp.int32)
    return boxes, labels, scores
